# Initial kernel scaffold; baseline (speedup 1.0000x reference)
import jax
import jax.numpy as jnp
from jax.experimental import pallas as pl


def kernel(x_materials, x_concepts, edge_index_mc, edge_label_index, Wl1_mc, bl1_mc, Wr1_mc, Wl1_cm, bl1_cm, Wr1_cm, Wl2_mc, bl2_mc, Wr2_mc, Wl2_cm, bl2_cm, Wr2_cm, W_lin1, b_lin1, W_lin2, b_lin2):
    raise NotImplementedError("write your pallas kernel here")


# speedup vs baseline: 3.4811x; 3.4811x over previous
import functools

import jax
import jax.numpy as jnp
from jax import lax
from jax.experimental import pallas as pl
from jax.experimental.pallas import tpu as pltpu
from jax.experimental.pallas import tpu_sc as plsc

N_MAT = 10000
N_CON = 10000
E = 320000
D = 128
H = 128
O = 64
L = 100000

NC = 2
NS = 16
CHUNK = 128
BLK = 1000

_HIGH = lax.Precision.HIGHEST


def _mesh():
    return plsc.VectorSubcoreMesh(
        core_axis_name="c", subcore_axis_name="s", num_cores=NC, num_subcores=NS
    )


def _make_agg(n_rows, width, n_edges, with_deg):
    n_chunks = n_edges // CHUNK
    assert n_edges % CHUNK == 0
    rpt = (n_rows // NS) // 8 * 8
    rtail = n_rows - rpt * NS

    out_type = [
        jax.ShapeDtypeStruct((n_rows, width), jnp.float32),
        jax.ShapeDtypeStruct((n_rows, width), jnp.float32),
    ]
    scratch = [
        pltpu.VMEM((CHUNK,), jnp.int32),
        pltpu.VMEM((CHUNK,), jnp.int32),
        pltpu.VMEM((CHUNK, width), jnp.float32),
        pltpu.VMEM_SHARED((n_rows, width), jnp.float32),
        pltpu.SemaphoreType.DMA,
    ]
    if with_deg:
        out_type += [
            jax.ShapeDtypeStruct((n_rows, width), jnp.float32),
            jax.ShapeDtypeStruct((n_rows, width), jnp.float32),
        ]
        scratch += [
            pltpu.VMEM((CHUNK, width), jnp.float32),
        ]

    def impl(xa, xb, src, dst, zrow, ones,
             acc_d, acc_s, deg_d, deg_s,
             gidx_v, sidx_v, rows_v, acc_sh, sem, ones_v):
        cid = lax.axis_index("c")
        sid = lax.axis_index("s")
        base = sid * rpt

        def zero_slabs():
            pltpu.sync_copy(zrow.at[pl.ds(base, rpt)], acc_sh.at[pl.ds(base, rpt)])
            if rtail:
                @pl.when(sid == NS - 1)
                def _():
                    tb = pl.ds(NS * rpt, rtail)
                    pltpu.sync_copy(zrow.at[tb], acc_sh.at[tb])

        zero_slabs()
        if with_deg:
            pltpu.sync_copy(ones, ones_v)
        plsc.subcore_barrier()

        def each_chunk(do_chunk):
            def chunk(i, carry):
                do_chunk(sid + i * NS)
                return carry

            n_even = n_chunks // NS
            lax.fori_loop(0, n_even, chunk, 0)
            if n_chunks % NS:
                @pl.when(sid < n_chunks % NS)
                def _():
                    do_chunk(sid + n_even * NS)

        def dump_slabs(out_hbm):
            pltpu.sync_copy(acc_sh.at[pl.ds(base, rpt)], out_hbm.at[pl.ds(base, rpt)])
            if rtail:
                @pl.when(sid == NS - 1)
                def _():
                    tb = pl.ds(NS * rpt, rtail)
                    pltpu.sync_copy(acc_sh.at[tb], out_hbm.at[tb])

        def run_all(x_hbm, g_hbm, s_hbm, acc_out, deg_out):
            def agg_chunk(j):
                b = j * CHUNK
                pltpu.sync_copy(g_hbm.at[pl.ds(b, CHUNK)], gidx_v)
                pltpu.sync_copy(s_hbm.at[pl.ds(b, CHUNK)], sidx_v)
                pltpu.async_copy(x_hbm.at[gidx_v], rows_v, sem).wait()
                pltpu.sync_copy(rows_v, acc_sh.at[sidx_v], add=True)

            each_chunk(agg_chunk)
            plsc.subcore_barrier()
            dump_slabs(acc_out)

            if with_deg:
                zero_slabs()
                plsc.subcore_barrier()

                def deg_chunk(j):
                    b = j * CHUNK
                    pltpu.sync_copy(s_hbm.at[pl.ds(b, CHUNK)], sidx_v)
                    pltpu.sync_copy(ones_v, acc_sh.at[sidx_v], add=True)

                each_chunk(deg_chunk)
                plsc.subcore_barrier()
                dump_slabs(deg_out)

        @pl.when(cid == 0)
        def _():
            run_all(xa, src, dst, acc_d, deg_d)

        @pl.when(cid == 1)
        def _():
            run_all(xb, dst, src, acc_s, deg_s)

    if with_deg:
        def body(xa, xb, src, dst, zrow, ones,
                 acc_d, acc_s, deg_d, deg_s,
                 gidx_v, sidx_v, rows_v, acc_sh, sem, ones_v):
            impl(xa, xb, src, dst, zrow, ones,
                 acc_d, acc_s, deg_d, deg_s,
                 gidx_v, sidx_v, rows_v, acc_sh, sem, ones_v)
    else:
        def body(xa, xb, src, dst, zrow,
                 acc_d, acc_s,
                 gidx_v, sidx_v, rows_v, acc_sh, sem):
            impl(xa, xb, src, dst, zrow, None,
                 acc_d, acc_s, None, None,
                 gidx_v, sidx_v, rows_v, acc_sh, sem, None)

    return pl.kernel(body, out_type=out_type, mesh=_mesh(), scratch_types=scratch)


def _make_pair_gather(n_out, width):
    n_full = n_out // CHUNK
    tail = n_out - n_full * CHUNK
    tail_tile = n_full % NS

    out_type = [
        jax.ShapeDtypeStruct((n_out, width), jnp.float32),
        jax.ShapeDtypeStruct((n_out, width), jnp.float32),
    ]
    scratch = [
        pltpu.VMEM((CHUNK,), jnp.int32),
        pltpu.VMEM((CHUNK, width), jnp.float32),
        pltpu.SemaphoreType.DMA,
    ]
    if tail:
        scratch += [
            pltpu.VMEM((tail,), jnp.int32),
            pltpu.VMEM((tail, width), jnp.float32),
        ]

    def impl(za, zb, ridx, cidx, oa, ob, idx_v, rows_v, sem, tidx_v, trows_v):
        cid = lax.axis_index("c")
        sid = lax.axis_index("s")

        def run(z_hbm, i_hbm, o_hbm):
            def do_chunk(j):
                b = j * CHUNK
                pltpu.sync_copy(i_hbm.at[pl.ds(b, CHUNK)], idx_v)
                pltpu.async_copy(z_hbm.at[idx_v], rows_v, sem).wait()
                pltpu.sync_copy(rows_v, o_hbm.at[pl.ds(b, CHUNK)])

            def chunk(i, carry):
                do_chunk(sid + i * NS)
                return carry

            n_even = n_full // NS
            lax.fori_loop(0, n_even, chunk, 0)
            if n_full % NS:
                @pl.when(sid < n_full % NS)
                def _():
                    do_chunk(sid + n_even * NS)
            if tail:
                @pl.when(sid == tail_tile)
                def _():
                    b = n_full * CHUNK
                    pltpu.sync_copy(i_hbm.at[pl.ds(b, tail)], tidx_v)
                    pltpu.async_copy(z_hbm.at[tidx_v], trows_v, sem).wait()
                    pltpu.sync_copy(trows_v, o_hbm.at[pl.ds(b, tail)])

        @pl.when(cid == 0)
        def _():
            run(za, ridx, oa)

        @pl.when(cid == 1)
        def _():
            run(zb, cidx, ob)

    if tail:
        def body(za, zb, ridx, cidx, oa, ob, idx_v, rows_v, sem, tidx_v, trows_v):
            impl(za, zb, ridx, cidx, oa, ob, idx_v, rows_v, sem, tidx_v, trows_v)
    else:
        def body(za, zb, ridx, cidx, oa, ob, idx_v, rows_v, sem):
            impl(za, zb, ridx, cidx, oa, ob, idx_v, rows_v, sem, None, None)

    return pl.kernel(body, out_type=out_type, mesh=_mesh(), scratch_types=scratch)


def _dot(a, b):
    return lax.dot_general(a, b, (((1,), (0,)), ((), ())), precision=_HIGH,
                           preferred_element_type=jnp.float32)


def _dot_t(a, b):
    return lax.dot_general(a, b, (((1,), (1,)), ((), ())), precision=_HIGH,
                           preferred_element_type=jnp.float32)


def _cnt(deg_ref):
    return jnp.maximum(deg_ref[:, 0:1], 1.0)


def _l1_body(acc_ref, deg_ref, x_ref, wl1_ref, wr1_ref, wr2_ref,
             bl1_ref, bl2_ref, h_ref, r_ref):
    mean = acc_ref[...] / _cnt(deg_ref)
    h = _dot(mean, wl1_ref[...]) + bl1_ref[...] + _dot(x_ref[...], wr1_ref[...])
    h = jnp.maximum(h, 0.0)
    h_ref[...] = h
    r_ref[...] = _dot(h, wr2_ref[...]) + bl2_ref[...]


def _layer1(acc, degs, x, wl1, wr1, wr2_dst, bl1, bl2_dst):
    n = acc.shape[0]
    grid = n // BLK
    full = lambda s: pl.BlockSpec(s, lambda i: (0, 0))
    rows = lambda w: pl.BlockSpec((BLK, w), lambda i: (i, 0))
    return pl.pallas_call(
        _l1_body,
        grid=(grid,),
        in_specs=[rows(H), rows(H), rows(D), full((D, H)), full((D, H)),
                  full((H, O)), full((1, H)), full((1, O))],
        out_specs=[rows(H), rows(O)],
        out_shape=[jax.ShapeDtypeStruct((n, H), jnp.float32),
                   jax.ShapeDtypeStruct((n, O), jnp.float32)],
    )(acc, degs, x, wl1, wr1, wr2_dst, bl1.reshape(1, H), bl2_dst.reshape(1, O))


def _zfin_body(ac_ref, dc_ref, rc_ref, am_ref, dm_ref, rm_ref,
               wl2mc_ref, wl2cm_ref, zc_ref, zm_ref, z2_ref):
    mean_c = ac_ref[...] / _cnt(dc_ref)
    mean_m = am_ref[...] / _cnt(dm_ref)
    zc = _dot(mean_c, wl2mc_ref[...]) + rc_ref[...]
    zm = _dot(mean_m, wl2cm_ref[...]) + rm_ref[...]
    zc_ref[...] = zc
    zm_ref[...] = zm
    z2_ref[...] = jnp.concatenate([zm, zc], axis=1)


def _z_finalize(acc2_c, degs_c, r_c, acc2_m, degs_m, r_m, wl2_mc, wl2_cm):
    n = acc2_c.shape[0]
    full = lambda s: pl.BlockSpec(s, lambda i: (0, 0))
    rows = lambda w: pl.BlockSpec((BLK, w), lambda i: (i, 0))
    return pl.pallas_call(
        _zfin_body,
        grid=(n // BLK,),
        in_specs=[rows(H), rows(H), rows(O), rows(H), rows(H), rows(O),
                  full((H, O)), full((H, O))],
        out_specs=[rows(O), rows(O), rows(2 * O)],
        out_shape=[jax.ShapeDtypeStruct((n, O), jnp.float32),
                   jax.ShapeDtypeStruct((n, O), jnp.float32),
                   jax.ShapeDtypeStruct((n, 2 * O), jnp.float32)],
    )(acc2_c, degs_c, r_c, acc2_m, degs_m, r_m, wl2_mc, wl2_cm)


def _dec_body(gr_ref, gc_ref, w1_ref, b1_ref, w2_ref, b2_ref, k_ref, probs_ref):
    kblk = jnp.concatenate([gr_ref[:, 0:O], gc_ref[:, O:2 * O]], axis=1)
    k_ref[...] = kblk
    hdec = jnp.maximum(_dot_t(kblk, w1_ref[...]) + b1_ref[...], 0.0)
    logits = _dot_t(hdec, w2_ref[...]) + b2_ref[...]
    l0 = logits[:, 0:1]
    l1 = logits[:, 1:2]
    m = jnp.maximum(l0, l1)
    e0 = jnp.exp(l0 - m)
    e1 = jnp.exp(l1 - m)
    s = e0 + e1
    probs_ref[...] = jnp.concatenate([e0 / s, e1 / s], axis=1)


def _decoder(g_r, g_c, w_lin1, b_lin1, w_lin2, b_lin2):
    blk = 1000
    full = lambda s: pl.BlockSpec(s, lambda i: (0, 0))
    rows = lambda w: pl.BlockSpec((blk, w), lambda i: (i, 0))
    return pl.pallas_call(
        _dec_body,
        grid=(L // blk,),
        in_specs=[rows(2 * O), rows(2 * O), full((O, 2 * O)), full((1, O)),
                  full((2, O)), full((1, 2))],
        out_specs=[rows(2 * O), rows(2)],
        out_shape=[jax.ShapeDtypeStruct((L, 2 * O), jnp.float32),
                   jax.ShapeDtypeStruct((L, 2), jnp.float32)],
    )(g_r, g_c, w_lin1, b_lin1.reshape(1, O), w_lin2, b_lin2.reshape(1, 2))


def kernel(x_materials, x_concepts, edge_index_mc, edge_label_index,
           Wl1_mc, bl1_mc, Wr1_mc, Wl1_cm, bl1_cm, Wr1_cm,
           Wl2_mc, bl2_mc, Wr2_mc, Wl2_cm, bl2_cm, Wr2_cm,
           W_lin1, b_lin1, W_lin2, b_lin2):
    src = edge_index_mc[0]
    dst = edge_index_mc[1]
    row = edge_label_index[0]
    col = edge_label_index[1]

    z128 = jnp.zeros((N_CON, D), jnp.float32)
    ones128 = jnp.ones((CHUNK, D), jnp.float32)

    agg_deg = _make_agg(N_CON, D, E, with_deg=True)
    acc1_c, acc1_m, deg_c, deg_m = agg_deg(
        x_materials, x_concepts, src, dst, z128, ones128)

    h_con, r_c = _layer1(acc1_c, deg_c, x_concepts,
                         Wl1_mc, Wr1_mc, Wr2_mc, bl1_mc, bl2_mc)
    h_mat, r_m = _layer1(acc1_m, deg_m, x_materials,
                         Wl1_cm, Wr1_cm, Wr2_cm, bl1_cm, bl2_cm)

    agg2 = _make_agg(N_CON, D, E, with_deg=False)
    acc2_c, acc2_m = agg2(h_mat, h_con, src, dst, z128)

    z_con, z_mat, z2 = _z_finalize(acc2_c, deg_c, r_c, acc2_m, deg_m, r_m,
                                   Wl2_mc, Wl2_cm)

    gather = _make_pair_gather(L, 2 * O)
    g_r, g_c = gather(z2, z2, row, col)

    k, probs = _decoder(g_r, g_c, W_lin1, b_lin1, W_lin2, b_lin2)
    return (probs, z_mat, z_con, k)


# speedup vs baseline: 4.5580x; 1.3093x over previous
import functools

import jax
import jax.numpy as jnp
from jax import lax
from jax.experimental import pallas as pl
from jax.experimental.pallas import tpu as pltpu
from jax.experimental.pallas import tpu_sc as plsc

N_MAT = 10000
N_CON = 10000
E = 320000
D = 128
H = 128
O = 64
L = 100000

NC = 2
NS = 16
CHUNK = 128
BLK = 1000

_HIGH = lax.Precision.HIGHEST


def _mesh():
    return plsc.VectorSubcoreMesh(
        core_axis_name="c", subcore_axis_name="s", num_cores=NC, num_subcores=NS
    )


def _make_agg(n_rows, width, n_edges, with_deg):
    n_chunks = n_edges // CHUNK
    assert n_edges % CHUNK == 0
    rpt = (n_rows // NS) // 8 * 8
    rtail = n_rows - rpt * NS

    out_type = [
        jax.ShapeDtypeStruct((n_rows, width), jnp.float32),
        jax.ShapeDtypeStruct((n_rows, width), jnp.float32),
    ]
    KB = 2 if with_deg else 3
    n_even = n_chunks // NS
    n_groups = n_even // KB
    n_loose = n_even - n_groups * KB

    scratch = (
        [pltpu.VMEM((CHUNK,), jnp.int32) for _ in range(KB)]
        + [pltpu.VMEM((CHUNK,), jnp.int32) for _ in range(KB)]
        + [pltpu.VMEM((CHUNK, width), jnp.float32) for _ in range(KB)]
        + [pltpu.VMEM_SHARED((n_rows, width), jnp.float32),
           pltpu.SemaphoreType.DMA,
           pltpu.SemaphoreType.DMA,
           pltpu.SemaphoreType.DMA]
    )
    if with_deg:
        out_type += [
            jax.ShapeDtypeStruct((n_rows, width), jnp.float32),
            jax.ShapeDtypeStruct((n_rows, width), jnp.float32),
        ]

    def impl(xa, xb, src, dst, zrow, ones,
             acc_d, acc_s, deg_d, deg_s,
             gidx, sidx, rows, acc_sh, sem_i, sem_g, sem_s):
        ones_v = rows[0]
        cid = lax.axis_index("c")
        sid = lax.axis_index("s")
        base = sid * rpt

        def zero_slabs():
            pltpu.sync_copy(zrow.at[pl.ds(base, rpt)], acc_sh.at[pl.ds(base, rpt)])
            if rtail:
                @pl.when(sid == NS - 1)
                def _():
                    tb = pl.ds(NS * rpt, rtail)
                    pltpu.sync_copy(zrow.at[tb], acc_sh.at[tb])

        zero_slabs()
        plsc.subcore_barrier()

        def dump_slabs(out_hbm):
            pltpu.sync_copy(acc_sh.at[pl.ds(base, rpt)], out_hbm.at[pl.ds(base, rpt)])
            if rtail:
                @pl.when(sid == NS - 1)
                def _():
                    tb = pl.ds(NS * rpt, rtail)
                    pltpu.sync_copy(acc_sh.at[tb], out_hbm.at[tb])

        def run_all(x_hbm, g_hbm, s_hbm, acc_out, deg_out):
            def agg_group(g, carry):
                j0 = sid + g * KB * NS
                dsc = []
                for b in range(KB):
                    bb = (j0 + b * NS) * CHUNK
                    dsc.append(pltpu.async_copy(
                        g_hbm.at[pl.ds(bb, CHUNK)], gidx[b], sem_i))
                    dsc.append(pltpu.async_copy(
                        s_hbm.at[pl.ds(bb, CHUNK)], sidx[b], sem_i))
                for d in dsc:
                    d.wait()
                dsc = [pltpu.async_copy(x_hbm.at[gidx[b]], rows[b], sem_g)
                       for b in range(KB)]
                for d in dsc:
                    d.wait()
                dsc = [pltpu.async_copy(rows[b], acc_sh.at[sidx[b]], sem_s,
                                        add=True)
                       for b in range(KB)]
                for d in dsc:
                    d.wait()
                return carry

            lax.fori_loop(0, n_groups, agg_group, 0)

            def agg_chunk(j):
                b = j * CHUNK
                pltpu.sync_copy(g_hbm.at[pl.ds(b, CHUNK)], gidx[0])
                pltpu.sync_copy(s_hbm.at[pl.ds(b, CHUNK)], sidx[0])
                pltpu.async_copy(x_hbm.at[gidx[0]], rows[0], sem_g).wait()
                pltpu.sync_copy(rows[0], acc_sh.at[sidx[0]], add=True)

            for t in range(n_loose):
                agg_chunk(sid + (n_groups * KB + t) * NS)
            if n_chunks % NS:
                @pl.when(sid < n_chunks % NS)
                def _():
                    agg_chunk(sid + n_even * NS)

            plsc.subcore_barrier()
            dump_slabs(acc_out)

            if with_deg:
                zero_slabs()
                pltpu.sync_copy(ones, ones_v)
                plsc.subcore_barrier()

                def deg_group(g, carry):
                    j0 = sid + g * KB * NS
                    dsc = []
                    for b in range(KB):
                        bb = (j0 + b * NS) * CHUNK
                        dsc.append(pltpu.async_copy(
                            s_hbm.at[pl.ds(bb, CHUNK)], sidx[b], sem_i))
                    for d in dsc:
                        d.wait()
                    dsc = [pltpu.async_copy(ones_v, acc_sh.at[sidx[b]], sem_s,
                                            add=True)
                           for b in range(KB)]
                    for d in dsc:
                        d.wait()
                    return carry

                lax.fori_loop(0, n_groups, deg_group, 0)

                def deg_chunk(j):
                    b = j * CHUNK
                    pltpu.sync_copy(s_hbm.at[pl.ds(b, CHUNK)], sidx[0])
                    pltpu.sync_copy(ones_v, acc_sh.at[sidx[0]], add=True)

                for t in range(n_loose):
                    deg_chunk(sid + (n_groups * KB + t) * NS)
                if n_chunks % NS:
                    @pl.when(sid < n_chunks % NS)
                    def _():
                        deg_chunk(sid + n_even * NS)

                plsc.subcore_barrier()
                dump_slabs(deg_out)

        @pl.when(cid == 0)
        def _():
            run_all(xa, src, dst, acc_d, deg_d)

        @pl.when(cid == 1)
        def _():
            run_all(xb, dst, src, acc_s, deg_s)

    if with_deg:
        def body(xa, xb, src, dst, zrow, ones,
                 acc_d, acc_s, deg_d, deg_s, *refs):
            gidx = refs[0:KB]
            sidx = refs[KB:2 * KB]
            rows = refs[2 * KB:3 * KB]
            acc_sh, sem_i, sem_g, sem_s = refs[3 * KB:]
            impl(xa, xb, src, dst, zrow, ones,
                 acc_d, acc_s, deg_d, deg_s,
                 gidx, sidx, rows, acc_sh, sem_i, sem_g, sem_s)
    else:
        def body(xa, xb, src, dst, zrow,
                 acc_d, acc_s, *refs):
            gidx = refs[0:KB]
            sidx = refs[KB:2 * KB]
            rows = refs[2 * KB:3 * KB]
            acc_sh, sem_i, sem_g, sem_s = refs[3 * KB:]
            impl(xa, xb, src, dst, zrow, None,
                 acc_d, acc_s, None, None,
                 gidx, sidx, rows, acc_sh, sem_i, sem_g, sem_s)

    return pl.kernel(body, out_type=out_type, mesh=_mesh(), scratch_types=scratch)


def _make_pair_gather(n_out, width):
    n_full = n_out // CHUNK
    tail = n_out - n_full * CHUNK
    tail_tile = n_full % NS

    KB = 6
    n_even = n_full // NS
    n_groups = n_even // KB
    n_loose = n_even - n_groups * KB

    out_type = [
        jax.ShapeDtypeStruct((n_out, width), jnp.float32),
        jax.ShapeDtypeStruct((n_out, width), jnp.float32),
    ]
    scratch = (
        [pltpu.VMEM((CHUNK,), jnp.int32) for _ in range(KB)]
        + [pltpu.VMEM((CHUNK, width), jnp.float32) for _ in range(KB)]
        + [pltpu.SemaphoreType.DMA, pltpu.SemaphoreType.DMA,
           pltpu.SemaphoreType.DMA]
    )
    if tail:
        scratch += [
            pltpu.VMEM((tail,), jnp.int32),
            pltpu.VMEM((tail, width), jnp.float32),
        ]

    def impl(za, zb, ridx, cidx, oa, ob, idx, rows,
             sem_i, sem_g, sem_s, tidx_v, trows_v):
        cid = lax.axis_index("c")
        sid = lax.axis_index("s")

        def run(z_hbm, i_hbm, o_hbm):
            def group(g, carry):
                j0 = sid + g * KB * NS
                dsc = []
                for b in range(KB):
                    bb = (j0 + b * NS) * CHUNK
                    dsc.append(pltpu.async_copy(
                        i_hbm.at[pl.ds(bb, CHUNK)], idx[b], sem_i))
                for d in dsc:
                    d.wait()
                dsc = [pltpu.async_copy(z_hbm.at[idx[b]], rows[b], sem_g)
                       for b in range(KB)]
                for d in dsc:
                    d.wait()
                dsc = []
                for b in range(KB):
                    bb = (j0 + b * NS) * CHUNK
                    dsc.append(pltpu.async_copy(
                        rows[b], o_hbm.at[pl.ds(bb, CHUNK)], sem_s))
                for d in dsc:
                    d.wait()
                return carry

            lax.fori_loop(0, n_groups, group, 0)

            def do_chunk(j):
                b = j * CHUNK
                pltpu.sync_copy(i_hbm.at[pl.ds(b, CHUNK)], idx[0])
                pltpu.async_copy(z_hbm.at[idx[0]], rows[0], sem_g).wait()
                pltpu.sync_copy(rows[0], o_hbm.at[pl.ds(b, CHUNK)])

            for t in range(n_loose):
                do_chunk(sid + (n_groups * KB + t) * NS)
            if n_full % NS:
                @pl.when(sid < n_full % NS)
                def _():
                    do_chunk(sid + n_even * NS)
            if tail:
                @pl.when(sid == tail_tile)
                def _():
                    b = n_full * CHUNK
                    pltpu.sync_copy(i_hbm.at[pl.ds(b, tail)], tidx_v)
                    pltpu.async_copy(z_hbm.at[tidx_v], trows_v, sem_g).wait()
                    pltpu.sync_copy(trows_v, o_hbm.at[pl.ds(b, tail)])

        @pl.when(cid == 0)
        def _():
            run(za, ridx, oa)

        @pl.when(cid == 1)
        def _():
            run(zb, cidx, ob)

    if tail:
        def body(za, zb, ridx, cidx, oa, ob, *refs):
            idx = refs[0:KB]
            rows = refs[KB:2 * KB]
            sem_i, sem_g, sem_s, tidx_v, trows_v = refs[2 * KB:]
            impl(za, zb, ridx, cidx, oa, ob, idx, rows,
                 sem_i, sem_g, sem_s, tidx_v, trows_v)
    else:
        def body(za, zb, ridx, cidx, oa, ob, *refs):
            idx = refs[0:KB]
            rows = refs[KB:2 * KB]
            sem_i, sem_g, sem_s = refs[2 * KB:]
            impl(za, zb, ridx, cidx, oa, ob, idx, rows,
                 sem_i, sem_g, sem_s, None, None)

    return pl.kernel(body, out_type=out_type, mesh=_mesh(), scratch_types=scratch)


def _dot(a, b):
    return lax.dot_general(a, b, (((1,), (0,)), ((), ())), precision=_HIGH,
                           preferred_element_type=jnp.float32)


def _dot_t(a, b):
    return lax.dot_general(a, b, (((1,), (1,)), ((), ())), precision=_HIGH,
                           preferred_element_type=jnp.float32)


def _cnt(deg_ref):
    return jnp.maximum(deg_ref[:, 0:1], 1.0)


def _l1_body(acc_ref, deg_ref, x_ref, wl1_ref, wr1_ref, wr2_ref,
             bl1_ref, bl2_ref, h_ref, r_ref):
    mean = acc_ref[...] / _cnt(deg_ref)
    h = _dot(mean, wl1_ref[...]) + bl1_ref[...] + _dot(x_ref[...], wr1_ref[...])
    h = jnp.maximum(h, 0.0)
    h_ref[...] = h
    r_ref[...] = _dot(h, wr2_ref[...]) + bl2_ref[...]


def _layer1(acc, degs, x, wl1, wr1, wr2_dst, bl1, bl2_dst):
    n = acc.shape[0]
    grid = n // BLK
    full = lambda s: pl.BlockSpec(s, lambda i: (0, 0))
    rows = lambda w: pl.BlockSpec((BLK, w), lambda i: (i, 0))
    return pl.pallas_call(
        _l1_body,
        grid=(grid,),
        in_specs=[rows(H), rows(H), rows(D), full((D, H)), full((D, H)),
                  full((H, O)), full((1, H)), full((1, O))],
        out_specs=[rows(H), rows(O)],
        out_shape=[jax.ShapeDtypeStruct((n, H), jnp.float32),
                   jax.ShapeDtypeStruct((n, O), jnp.float32)],
    )(acc, degs, x, wl1, wr1, wr2_dst, bl1.reshape(1, H), bl2_dst.reshape(1, O))


def _zfin_body(ac_ref, dc_ref, rc_ref, am_ref, dm_ref, rm_ref,
               wl2mc_ref, wl2cm_ref, zc_ref, zm_ref, z2_ref):
    mean_c = ac_ref[...] / _cnt(dc_ref)
    mean_m = am_ref[...] / _cnt(dm_ref)
    zc = _dot(mean_c, wl2mc_ref[...]) + rc_ref[...]
    zm = _dot(mean_m, wl2cm_ref[...]) + rm_ref[...]
    zc_ref[...] = zc
    zm_ref[...] = zm
    z2_ref[...] = jnp.concatenate([zm, zc], axis=1)


def _z_finalize(acc2_c, degs_c, r_c, acc2_m, degs_m, r_m, wl2_mc, wl2_cm):
    n = acc2_c.shape[0]
    full = lambda s: pl.BlockSpec(s, lambda i: (0, 0))
    rows = lambda w: pl.BlockSpec((BLK, w), lambda i: (i, 0))
    return pl.pallas_call(
        _zfin_body,
        grid=(n // BLK,),
        in_specs=[rows(H), rows(H), rows(O), rows(H), rows(H), rows(O),
                  full((H, O)), full((H, O))],
        out_specs=[rows(O), rows(O), rows(2 * O)],
        out_shape=[jax.ShapeDtypeStruct((n, O), jnp.float32),
                   jax.ShapeDtypeStruct((n, O), jnp.float32),
                   jax.ShapeDtypeStruct((n, 2 * O), jnp.float32)],
    )(acc2_c, degs_c, r_c, acc2_m, degs_m, r_m, wl2_mc, wl2_cm)


def _dec_body(gr_ref, gc_ref, w1_ref, b1_ref, w2_ref, b2_ref, k_ref, probs_ref):
    kblk = jnp.concatenate([gr_ref[:, 0:O], gc_ref[:, O:2 * O]], axis=1)
    k_ref[...] = kblk
    hdec = jnp.maximum(_dot_t(kblk, w1_ref[...]) + b1_ref[...], 0.0)
    logits = _dot_t(hdec, w2_ref[...]) + b2_ref[...]
    l0 = logits[:, 0:1]
    l1 = logits[:, 1:2]
    m = jnp.maximum(l0, l1)
    e0 = jnp.exp(l0 - m)
    e1 = jnp.exp(l1 - m)
    s = e0 + e1
    probs_ref[...] = jnp.concatenate([e0 / s, e1 / s], axis=1)


def _decoder(g_r, g_c, w_lin1, b_lin1, w_lin2, b_lin2):
    blk = 1000
    full = lambda s: pl.BlockSpec(s, lambda i: (0, 0))
    rows = lambda w: pl.BlockSpec((blk, w), lambda i: (i, 0))
    return pl.pallas_call(
        _dec_body,
        grid=(L // blk,),
        in_specs=[rows(2 * O), rows(2 * O), full((O, 2 * O)), full((1, O)),
                  full((2, O)), full((1, 2))],
        out_specs=[rows(2 * O), rows(2)],
        out_shape=[jax.ShapeDtypeStruct((L, 2 * O), jnp.float32),
                   jax.ShapeDtypeStruct((L, 2), jnp.float32)],
    )(g_r, g_c, w_lin1, b_lin1.reshape(1, O), w_lin2, b_lin2.reshape(1, 2))


def kernel(x_materials, x_concepts, edge_index_mc, edge_label_index,
           Wl1_mc, bl1_mc, Wr1_mc, Wl1_cm, bl1_cm, Wr1_cm,
           Wl2_mc, bl2_mc, Wr2_mc, Wl2_cm, bl2_cm, Wr2_cm,
           W_lin1, b_lin1, W_lin2, b_lin2):
    src = edge_index_mc[0]
    dst = edge_index_mc[1]
    row = edge_label_index[0]
    col = edge_label_index[1]

    z128 = jnp.zeros((N_CON, D), jnp.float32)
    ones128 = jnp.ones((CHUNK, D), jnp.float32)

    agg_deg = _make_agg(N_CON, D, E, with_deg=True)
    acc1_c, acc1_m, deg_c, deg_m = agg_deg(
        x_materials, x_concepts, src, dst, z128, ones128)

    h_con, r_c = _layer1(acc1_c, deg_c, x_concepts,
                         Wl1_mc, Wr1_mc, Wr2_mc, bl1_mc, bl2_mc)
    h_mat, r_m = _layer1(acc1_m, deg_m, x_materials,
                         Wl1_cm, Wr1_cm, Wr2_cm, bl1_cm, bl2_cm)

    agg2 = _make_agg(N_CON, D, E, with_deg=False)
    acc2_c, acc2_m = agg2(h_mat, h_con, src, dst, z128)

    z_con, z_mat, z2 = _z_finalize(acc2_c, deg_c, r_c, acc2_m, deg_m, r_m,
                                   Wl2_mc, Wl2_cm)

    gather = _make_pair_gather(L, 2 * O)
    g_r, g_c = gather(z2, z2, row, col)

    k, probs = _decoder(g_r, g_c, W_lin1, b_lin1, W_lin2, b_lin2)
    return (probs, z_mat, z_con, k)


# speedup vs baseline: 4.6365x; 1.0172x over previous
import functools

import jax
import jax.numpy as jnp
from jax import lax
from jax.experimental import pallas as pl
from jax.experimental.pallas import tpu as pltpu
from jax.experimental.pallas import tpu_sc as plsc

N_MAT = 10000
N_CON = 10000
E = 320000
D = 128
H = 128
O = 64
L = 100000

NC = 2
NS = 16
CHUNK = 128
BLK = 1000

_HIGH = lax.Precision.HIGHEST


def _mesh():
    return plsc.VectorSubcoreMesh(
        core_axis_name="c", subcore_axis_name="s", num_cores=NC, num_subcores=NS
    )


def _make_agg(n_rows, width, n_edges, with_deg):
    n_chunks = n_edges // CHUNK
    assert n_edges % CHUNK == 0
    rpt = (n_rows // NS) // 8 * 8
    rtail = n_rows - rpt * NS

    out_type = [
        jax.ShapeDtypeStruct((n_rows, width), jnp.float32),
        jax.ShapeDtypeStruct((n_rows, width), jnp.float32),
    ]
    KB = 3
    n_even = n_chunks // NS
    n_groups = n_even // KB
    n_loose = n_even - n_groups * KB

    scratch = (
        [pltpu.VMEM((CHUNK,), jnp.int32) for _ in range(KB)]
        + [pltpu.VMEM((CHUNK,), jnp.int32) for _ in range(KB)]
        + [pltpu.VMEM((CHUNK, width), jnp.float32) for _ in range(KB)]
        + [pltpu.VMEM_SHARED((n_rows, width), jnp.float32),
           pltpu.SemaphoreType.DMA,
           pltpu.SemaphoreType.DMA,
           pltpu.SemaphoreType.DMA]
    )
    if with_deg:
        out_type += [
            jax.ShapeDtypeStruct((n_rows, width), jnp.float32),
            jax.ShapeDtypeStruct((n_rows, width), jnp.float32),
        ]

    def impl(xa, xb, src, dst, zrow, ones,
             acc_d, acc_s, deg_d, deg_s,
             gidx, sidx, rows, acc_sh, sem_i, sem_g, sem_s):
        ones_v = rows[0]
        cid = lax.axis_index("c")
        sid = lax.axis_index("s")
        base = sid * rpt

        def zero_slabs():
            pltpu.sync_copy(zrow.at[pl.ds(base, rpt)], acc_sh.at[pl.ds(base, rpt)])
            if rtail:
                @pl.when(sid == NS - 1)
                def _():
                    tb = pl.ds(NS * rpt, rtail)
                    pltpu.sync_copy(zrow.at[tb], acc_sh.at[tb])

        zero_slabs()
        plsc.subcore_barrier()

        def dump_slabs(out_hbm):
            pltpu.sync_copy(acc_sh.at[pl.ds(base, rpt)], out_hbm.at[pl.ds(base, rpt)])
            if rtail:
                @pl.when(sid == NS - 1)
                def _():
                    tb = pl.ds(NS * rpt, rtail)
                    pltpu.sync_copy(acc_sh.at[tb], out_hbm.at[tb])

        def run_all(x_hbm, g_hbm, s_hbm, acc_out, deg_out):
            def agg_group(g, carry):
                j0 = sid + g * KB * NS
                dsc = []
                for b in range(KB):
                    bb = (j0 + b * NS) * CHUNK
                    dsc.append(pltpu.async_copy(
                        g_hbm.at[pl.ds(bb, CHUNK)], gidx[b], sem_i))
                    dsc.append(pltpu.async_copy(
                        s_hbm.at[pl.ds(bb, CHUNK)], sidx[b], sem_i))
                for d in dsc:
                    d.wait()
                dsc = [pltpu.async_copy(x_hbm.at[gidx[b]], rows[b], sem_g)
                       for b in range(KB)]
                for d in dsc:
                    d.wait()
                dsc = [pltpu.async_copy(rows[b], acc_sh.at[sidx[b]], sem_s,
                                        add=True)
                       for b in range(KB)]
                for d in dsc:
                    d.wait()
                return carry

            lax.fori_loop(0, n_groups, agg_group, 0)

            def agg_chunk(j):
                b = j * CHUNK
                pltpu.sync_copy(g_hbm.at[pl.ds(b, CHUNK)], gidx[0])
                pltpu.sync_copy(s_hbm.at[pl.ds(b, CHUNK)], sidx[0])
                pltpu.async_copy(x_hbm.at[gidx[0]], rows[0], sem_g).wait()
                pltpu.sync_copy(rows[0], acc_sh.at[sidx[0]], add=True)

            for t in range(n_loose):
                agg_chunk(sid + (n_groups * KB + t) * NS)
            if n_chunks % NS:
                @pl.when(sid < n_chunks % NS)
                def _():
                    agg_chunk(sid + n_even * NS)

            plsc.subcore_barrier()
            dump_slabs(acc_out)

            if with_deg:
                zero_slabs()
                pltpu.sync_copy(ones, ones_v)
                plsc.subcore_barrier()

                def deg_group(g, carry):
                    j0 = sid + g * KB * NS
                    dsc = []
                    for b in range(KB):
                        bb = (j0 + b * NS) * CHUNK
                        dsc.append(pltpu.async_copy(
                            s_hbm.at[pl.ds(bb, CHUNK)], sidx[b], sem_i))
                    for d in dsc:
                        d.wait()
                    dsc = [pltpu.async_copy(ones_v, acc_sh.at[sidx[b]], sem_s,
                                            add=True)
                           for b in range(KB)]
                    for d in dsc:
                        d.wait()
                    return carry

                lax.fori_loop(0, n_groups, deg_group, 0)

                def deg_chunk(j):
                    b = j * CHUNK
                    pltpu.sync_copy(s_hbm.at[pl.ds(b, CHUNK)], sidx[0])
                    pltpu.sync_copy(ones_v, acc_sh.at[sidx[0]], add=True)

                for t in range(n_loose):
                    deg_chunk(sid + (n_groups * KB + t) * NS)
                if n_chunks % NS:
                    @pl.when(sid < n_chunks % NS)
                    def _():
                        deg_chunk(sid + n_even * NS)

                plsc.subcore_barrier()
                dump_slabs(deg_out)

        @pl.when(cid == 0)
        def _():
            run_all(xa, src, dst, acc_d, deg_d)

        @pl.when(cid == 1)
        def _():
            run_all(xb, dst, src, acc_s, deg_s)

    if with_deg:
        def body(xa, xb, src, dst, zrow, ones,
                 acc_d, acc_s, deg_d, deg_s, *refs):
            gidx = refs[0:KB]
            sidx = refs[KB:2 * KB]
            rows = refs[2 * KB:3 * KB]
            acc_sh, sem_i, sem_g, sem_s = refs[3 * KB:]
            impl(xa, xb, src, dst, zrow, ones,
                 acc_d, acc_s, deg_d, deg_s,
                 gidx, sidx, rows, acc_sh, sem_i, sem_g, sem_s)
    else:
        def body(xa, xb, src, dst, zrow,
                 acc_d, acc_s, *refs):
            gidx = refs[0:KB]
            sidx = refs[KB:2 * KB]
            rows = refs[2 * KB:3 * KB]
            acc_sh, sem_i, sem_g, sem_s = refs[3 * KB:]
            impl(xa, xb, src, dst, zrow, None,
                 acc_d, acc_s, None, None,
                 gidx, sidx, rows, acc_sh, sem_i, sem_g, sem_s)

    return pl.kernel(body, out_type=out_type, mesh=_mesh(), scratch_types=scratch)


def _make_pair_gather(n_out, width):
    n_full = n_out // CHUNK
    tail = n_out - n_full * CHUNK
    tail_tile = n_full % NS

    KB = 6
    n_even = n_full // NS
    n_groups = n_even // KB
    n_loose = n_even - n_groups * KB

    out_type = [
        jax.ShapeDtypeStruct((n_out, width), jnp.float32),
        jax.ShapeDtypeStruct((n_out, width), jnp.float32),
    ]
    scratch = (
        [pltpu.VMEM((CHUNK,), jnp.int32) for _ in range(KB)]
        + [pltpu.VMEM((CHUNK, width), jnp.float32) for _ in range(KB)]
        + [pltpu.SemaphoreType.DMA, pltpu.SemaphoreType.DMA,
           pltpu.SemaphoreType.DMA]
    )
    if tail:
        scratch += [
            pltpu.VMEM((tail,), jnp.int32),
            pltpu.VMEM((tail, width), jnp.float32),
        ]

    def impl(za, zb, ridx, cidx, oa, ob, idx, rows,
             sem_i, sem_g, sem_s, tidx_v, trows_v):
        cid = lax.axis_index("c")
        sid = lax.axis_index("s")

        def run(z_hbm, i_hbm, o_hbm):
            def group(g, carry):
                j0 = sid + g * KB * NS
                dsc = []
                for b in range(KB):
                    bb = (j0 + b * NS) * CHUNK
                    dsc.append(pltpu.async_copy(
                        i_hbm.at[pl.ds(bb, CHUNK)], idx[b], sem_i))
                for d in dsc:
                    d.wait()
                dsc = [pltpu.async_copy(z_hbm.at[idx[b]], rows[b], sem_g)
                       for b in range(KB)]
                for d in dsc:
                    d.wait()
                dsc = []
                for b in range(KB):
                    bb = (j0 + b * NS) * CHUNK
                    dsc.append(pltpu.async_copy(
                        rows[b], o_hbm.at[pl.ds(bb, CHUNK)], sem_s))
                for d in dsc:
                    d.wait()
                return carry

            lax.fori_loop(0, n_groups, group, 0)

            def do_chunk(j):
                b = j * CHUNK
                pltpu.sync_copy(i_hbm.at[pl.ds(b, CHUNK)], idx[0])
                pltpu.async_copy(z_hbm.at[idx[0]], rows[0], sem_g).wait()
                pltpu.sync_copy(rows[0], o_hbm.at[pl.ds(b, CHUNK)])

            for t in range(n_loose):
                do_chunk(sid + (n_groups * KB + t) * NS)
            if n_full % NS:
                @pl.when(sid < n_full % NS)
                def _():
                    do_chunk(sid + n_even * NS)
            if tail:
                @pl.when(sid == tail_tile)
                def _():
                    b = n_full * CHUNK
                    pltpu.sync_copy(i_hbm.at[pl.ds(b, tail)], tidx_v)
                    pltpu.async_copy(z_hbm.at[tidx_v], trows_v, sem_g).wait()
                    pltpu.sync_copy(trows_v, o_hbm.at[pl.ds(b, tail)])

        @pl.when(cid == 0)
        def _():
            run(za, ridx, oa)

        @pl.when(cid == 1)
        def _():
            run(zb, cidx, ob)

    if tail:
        def body(za, zb, ridx, cidx, oa, ob, *refs):
            idx = refs[0:KB]
            rows = refs[KB:2 * KB]
            sem_i, sem_g, sem_s, tidx_v, trows_v = refs[2 * KB:]
            impl(za, zb, ridx, cidx, oa, ob, idx, rows,
                 sem_i, sem_g, sem_s, tidx_v, trows_v)
    else:
        def body(za, zb, ridx, cidx, oa, ob, *refs):
            idx = refs[0:KB]
            rows = refs[KB:2 * KB]
            sem_i, sem_g, sem_s = refs[2 * KB:]
            impl(za, zb, ridx, cidx, oa, ob, idx, rows,
                 sem_i, sem_g, sem_s, None, None)

    return pl.kernel(body, out_type=out_type, mesh=_mesh(), scratch_types=scratch)


def _dot(a, b):
    return lax.dot_general(a, b, (((1,), (0,)), ((), ())), precision=_HIGH,
                           preferred_element_type=jnp.float32)


def _dot_t(a, b):
    return lax.dot_general(a, b, (((1,), (1,)), ((), ())), precision=_HIGH,
                           preferred_element_type=jnp.float32)


def _cnt(deg_ref):
    return jnp.maximum(deg_ref[:, 0:1], 1.0)


def _l1_pair(ref1, ref2, h_ref, r_ref):
    (acc_ref, deg_ref, x_ref, wl1_ref, wr1_ref, wr2_ref, bl1_ref, bl2_ref) = ref1
    mean = acc_ref[...] / _cnt(deg_ref)
    h = _dot(mean, wl1_ref[...]) + bl1_ref[...] + _dot(x_ref[...], wr1_ref[...])
    h = jnp.maximum(h, 0.0)
    h_ref[...] = h
    r_ref[...] = _dot(h, wr2_ref[...]) + bl2_ref[...]


def _l1_body(ac_ref, dc_ref, xc_ref, wl1c_ref, wr1c_ref, wr2c_ref,
             bl1c_ref, bl2c_ref,
             am_ref, dm_ref, xm_ref, wl1m_ref, wr1m_ref, wr2m_ref,
             bl1m_ref, bl2m_ref,
             hc_ref, rc_ref, hm_ref, rm_ref):
    _l1_pair((ac_ref, dc_ref, xc_ref, wl1c_ref, wr1c_ref, wr2c_ref,
              bl1c_ref, bl2c_ref), None, hc_ref, rc_ref)
    _l1_pair((am_ref, dm_ref, xm_ref, wl1m_ref, wr1m_ref, wr2m_ref,
              bl1m_ref, bl2m_ref), None, hm_ref, rm_ref)


def _layer1_both(args_c, args_m):
    n = args_c[0].shape[0]
    grid = n // BLK
    full = lambda s: pl.BlockSpec(s, lambda i: (0, 0))
    rows = lambda w: pl.BlockSpec((BLK, w), lambda i: (i, 0))
    type_specs = [rows(H), rows(H), rows(D), full((D, H)), full((D, H)),
                  full((H, O)), full((1, H)), full((1, O))]

    def prep(a):
        (acc, degs, x, wl1, wr1, wr2_dst, bl1, bl2_dst) = a
        return (acc, degs, x, wl1, wr1, wr2_dst,
                bl1.reshape(1, H), bl2_dst.reshape(1, O))

    return pl.pallas_call(
        _l1_body,
        grid=(grid,),
        in_specs=type_specs + type_specs,
        out_specs=[rows(H), rows(O), rows(H), rows(O)],
        out_shape=[jax.ShapeDtypeStruct((n, H), jnp.float32),
                   jax.ShapeDtypeStruct((n, O), jnp.float32),
                   jax.ShapeDtypeStruct((n, H), jnp.float32),
                   jax.ShapeDtypeStruct((n, O), jnp.float32)],
    )(*prep(args_c), *prep(args_m))


def _zfin_body(ac_ref, dc_ref, rc_ref, am_ref, dm_ref, rm_ref,
               wl2mc_ref, wl2cm_ref, zc_ref, zm_ref, z2_ref):
    mean_c = ac_ref[...] / _cnt(dc_ref)
    mean_m = am_ref[...] / _cnt(dm_ref)
    zc = _dot(mean_c, wl2mc_ref[...]) + rc_ref[...]
    zm = _dot(mean_m, wl2cm_ref[...]) + rm_ref[...]
    zc_ref[...] = zc
    zm_ref[...] = zm
    z2_ref[...] = jnp.concatenate([zm, zc], axis=1)


def _z_finalize(acc2_c, degs_c, r_c, acc2_m, degs_m, r_m, wl2_mc, wl2_cm):
    n = acc2_c.shape[0]
    full = lambda s: pl.BlockSpec(s, lambda i: (0, 0))
    rows = lambda w: pl.BlockSpec((BLK, w), lambda i: (i, 0))
    return pl.pallas_call(
        _zfin_body,
        grid=(n // BLK,),
        in_specs=[rows(H), rows(H), rows(O), rows(H), rows(H), rows(O),
                  full((H, O)), full((H, O))],
        out_specs=[rows(O), rows(O), rows(2 * O)],
        out_shape=[jax.ShapeDtypeStruct((n, O), jnp.float32),
                   jax.ShapeDtypeStruct((n, O), jnp.float32),
                   jax.ShapeDtypeStruct((n, 2 * O), jnp.float32)],
    )(acc2_c, degs_c, r_c, acc2_m, degs_m, r_m, wl2_mc, wl2_cm)


def _dec_body(gr_ref, gc_ref, w1_ref, b1_ref, w2_ref, b2_ref, k_ref, probs_ref):
    kblk = jnp.concatenate([gr_ref[:, 0:O], gc_ref[:, O:2 * O]], axis=1)
    k_ref[...] = kblk
    hdec = jnp.maximum(_dot_t(kblk, w1_ref[...]) + b1_ref[...], 0.0)
    logits = _dot_t(hdec, w2_ref[...]) + b2_ref[...]
    l0 = logits[:, 0:1]
    l1 = logits[:, 1:2]
    m = jnp.maximum(l0, l1)
    e0 = jnp.exp(l0 - m)
    e1 = jnp.exp(l1 - m)
    s = e0 + e1
    probs_ref[...] = jnp.concatenate([e0 / s, e1 / s], axis=1)


def _decoder(g_r, g_c, w_lin1, b_lin1, w_lin2, b_lin2):
    blk = 1000
    full = lambda s: pl.BlockSpec(s, lambda i: (0, 0))
    rows = lambda w: pl.BlockSpec((blk, w), lambda i: (i, 0))
    return pl.pallas_call(
        _dec_body,
        grid=(L // blk,),
        in_specs=[rows(2 * O), rows(2 * O), full((O, 2 * O)), full((1, O)),
                  full((2, O)), full((1, 2))],
        out_specs=[rows(2 * O), rows(2)],
        out_shape=[jax.ShapeDtypeStruct((L, 2 * O), jnp.float32),
                   jax.ShapeDtypeStruct((L, 2), jnp.float32)],
    )(g_r, g_c, w_lin1, b_lin1.reshape(1, O), w_lin2, b_lin2.reshape(1, 2))


def kernel(x_materials, x_concepts, edge_index_mc, edge_label_index,
           Wl1_mc, bl1_mc, Wr1_mc, Wl1_cm, bl1_cm, Wr1_cm,
           Wl2_mc, bl2_mc, Wr2_mc, Wl2_cm, bl2_cm, Wr2_cm,
           W_lin1, b_lin1, W_lin2, b_lin2):
    src = edge_index_mc[0]
    dst = edge_index_mc[1]
    row = edge_label_index[0]
    col = edge_label_index[1]

    z128 = jnp.zeros((N_CON, D), jnp.float32)
    ones128 = jnp.ones((CHUNK, D), jnp.float32)

    agg_deg = _make_agg(N_CON, D, E, with_deg=True)
    acc1_c, acc1_m, deg_c, deg_m = agg_deg(
        x_materials, x_concepts, src, dst, z128, ones128)

    h_con, r_c, h_mat, r_m = _layer1_both(
        (acc1_c, deg_c, x_concepts, Wl1_mc, Wr1_mc, Wr2_mc, bl1_mc, bl2_mc),
        (acc1_m, deg_m, x_materials, Wl1_cm, Wr1_cm, Wr2_cm, bl1_cm, bl2_cm))

    agg2 = _make_agg(N_CON, D, E, with_deg=False)
    acc2_c, acc2_m = agg2(h_mat, h_con, src, dst, z128)

    z_con, z_mat, z2 = _z_finalize(acc2_c, deg_c, r_c, acc2_m, deg_m, r_m,
                                   Wl2_mc, Wl2_cm)

    gather = _make_pair_gather(L, 2 * O)
    g_r, g_c = gather(z2, z2, row, col)

    k, probs = _decoder(g_r, g_c, W_lin1, b_lin1, W_lin2, b_lin2)
    return (probs, z_mat, z_con, k)
